# Initial kernel scaffold; baseline (speedup 1.0000x reference)
import jax
import jax.numpy as jnp
from jax.experimental import pallas as pl


def kernel(inputs, unnormalized_widths, unnormalized_heights, unnormalized_derivatives):
    raise NotImplementedError("write your pallas kernel here")


# speedup vs baseline: 18.2925x; 18.2925x over previous
import jax
import jax.numpy as jnp
from jax import lax
from jax.experimental import pallas as pl
from jax.experimental.pallas import tpu as pltpu
from jax.experimental.pallas import tpu_sc as plsc

BATCH = 8192
SHAPE = 512
NUM_BINS = 10
TAIL = 10.0
MIN_W = 1e-3
MIN_H = 1e-3
MIN_D = 1e-3

NC = 2
NS = 16
L = 16
NW = NC * NS
ROWS_W = BATCH // NW
BLK = 64
NBLK = ROWS_W // BLK
NG = SHAPE // L

_LN2 = 0.6931471805599453
_SQRT2 = 1.4142135623730951


def _ln(x):
    bits = plsc.bitcast(x, jnp.int32)
    e = (bits >> 23) - 127
    m = plsc.bitcast((bits & 0x007FFFFF) | 0x3F800000, jnp.float32)
    big = m >= _SQRT2
    m = jnp.where(big, m * 0.5, m)
    e = jnp.where(big, e + 1, e)
    t = (m - 1.0) / (m + 1.0)
    u = t * t
    poly = 1.0 + u * (1.0 / 3.0 + u * (1.0 / 5.0 + u * (1.0 / 7.0 + u * (1.0 / 9.0))))
    return e.astype(jnp.float32) * _LN2 + (2.0 * t) * poly


def _softplus(x):
    return jnp.maximum(x, 0.0) + _ln(1.0 + jnp.exp(-jnp.abs(x)))


def _body(x_hbm, uw_hbm, uh_hbm, ud_hbm, out_hbm, ld_hbm,
          uw_v, uh_v, ud_v, cw_v, rw_v, ch_v, dd_v, dl_v, hh_v,
          in_v, out_v, ldp_v, ld_v):
    wid = lax.axis_index("s") * NC + lax.axis_index("c")
    base = wid * ROWS_W

    pltpu.sync_copy(uw_hbm, uw_v)
    pltpu.sync_copy(uh_hbm, uh_v)
    pltpu.sync_copy(ud_hbm, ud_v)

    iota = lax.iota(jnp.int32, L)

    def _norm_cum(vals):
        m = vals[0]
        for v in vals[1:]:
            m = jnp.maximum(m, v)
        es = [jnp.exp(v - m) for v in vals]
        s = es[0]
        for v in es[1:]:
            s = s + v
        rs = 1.0 / s
        knots = [jnp.full((L,), -TAIL, jnp.float32)]
        c = jnp.zeros((L,), jnp.float32)
        for k in range(NUM_BINS):
            w = MIN_W + (1.0 - MIN_W * NUM_BINS) * (es[k] * rs)
            c = c + w
            if k == NUM_BINS - 1:
                knots.append(jnp.full((L,), TAIL, jnp.float32))
            else:
                knots.append(2.0 * TAIL * c - TAIL)
        return knots

    def _prep(g, _):
        c0 = g * L
        cols = iota + c0

        def par(ref, k, nb):
            return plsc.load_gather(ref, [cols * nb + k])

        cw = _norm_cum([par(uw_v, k, NUM_BINS) for k in range(NUM_BINS)])
        ch = _norm_cum([par(uh_v, k, NUM_BINS) for k in range(NUM_BINS)])
        for k in range(NUM_BINS + 1):
            cw_v[pl.ds(k * SHAPE + c0, L)] = cw[k]
            ch_v[pl.ds(k * SHAPE + c0, L)] = ch[k]
        for k in range(NUM_BINS):
            w = cw[k + 1] - cw[k]
            h = ch[k + 1] - ch[k]
            rw = 1.0 / w
            rw_v[pl.ds(k * SHAPE + c0, L)] = rw
            hh_v[pl.ds(k * SHAPE + c0, L)] = h
            dl_v[pl.ds(k * SHAPE + c0, L)] = h * rw
        one = jnp.full((L,), 1.0, jnp.float32)
        dd_v[pl.ds(c0, L)] = one
        dd_v[pl.ds(NUM_BINS * SHAPE + c0, L)] = one
        for k in range(1, NUM_BINS):
            d = MIN_D + _softplus(par(ud_v, k - 1, NUM_BINS - 1))
            dd_v[pl.ds(k * SHAPE + c0, L)] = d
        return 0

    lax.fori_loop(0, NG, _prep, 0)

    def _block(b, _):
        r0 = base + b * BLK
        pltpu.sync_copy(x_hbm.at[pl.ds(r0, BLK), :], in_v)

        def _row(r, _):
            def _group(g, acc):
                c0 = g * L
                x = in_v[r, pl.ds(c0, L)]
                xc = jnp.minimum(jnp.maximum(x, -TAIL), TAIL)
                bin_ = jnp.zeros((L,), jnp.int32)
                for k in range(1, NUM_BINS):
                    bin_ = bin_ + jnp.where(xc >= cw_v[pl.ds(k * SHAPE + c0, L)], 1, 0)
                idx = (bin_ << 9) + (iota + c0)
                cwb = plsc.load_gather(cw_v, [idx])
                rwb = plsc.load_gather(rw_v, [idx])
                chb = plsc.load_gather(ch_v, [idx])
                dlb = plsc.load_gather(dl_v, [idx])
                db = plsc.load_gather(dd_v, [idx])
                dpb = plsc.load_gather(dd_v, [idx + SHAPE])
                hb = plsc.load_gather(hh_v, [idx])

                th = (xc - cwb) * rwb
                om = 1.0 - th
                u = th * om
                th2 = th * th
                dl2 = dlb + dlb
                num = hb * (dlb * th2 + db * u)
                den = dlb + (db + dpb - dl2) * u
                rden = 1.0 / den
                out_s = chb + num * rden
                dnum = (dlb * dlb) * (dpb * th2 + dl2 * u + db * (om * om))
                lad = _ln(dnum * rden * rden)

                inside = (x >= -TAIL) & (x <= TAIL)
                out_v[r, pl.ds(c0, L)] = jnp.where(inside, out_s, x)
                return acc + jnp.where(inside, lad, 0.0)

            acc = lax.fori_loop(0, NG, _group, jnp.zeros((L,), jnp.float32))
            ldp_v[pl.ds(r * L, L)] = acc
            return 0

        lax.fori_loop(0, BLK, _row, 0)

        def _ldred(rr, _):
            rows16 = (iota + rr * L) << 4
            tot = plsc.load_gather(ldp_v, [rows16])
            for c in range(1, L):
                tot = tot + plsc.load_gather(ldp_v, [rows16 + c])
            ld_v[pl.ds(b * BLK + rr * L, L)] = tot
            return 0

        lax.fori_loop(0, BLK // L, _ldred, 0)
        pltpu.sync_copy(out_v, out_hbm.at[pl.ds(r0, BLK), :])
        return 0

    lax.fori_loop(0, NBLK, _block, 0)
    pltpu.sync_copy(ld_v, ld_hbm.at[pl.ds(base, ROWS_W)])


@jax.jit
def kernel(inputs, unnormalized_widths, unnormalized_heights,
           unnormalized_derivatives):
    mesh = plsc.VectorSubcoreMesh(core_axis_name="c", subcore_axis_name="s")
    f = pl.kernel(
        _body,
        out_type=(
            jax.ShapeDtypeStruct((BATCH, SHAPE), jnp.float32),
            jax.ShapeDtypeStruct((BATCH,), jnp.float32),
        ),
        mesh=mesh,
        compiler_params=pltpu.CompilerParams(needs_layout_passes=False),
        scratch_types=[
            pltpu.VMEM((SHAPE * NUM_BINS,), jnp.float32),
            pltpu.VMEM((SHAPE * NUM_BINS,), jnp.float32),
            pltpu.VMEM((SHAPE * (NUM_BINS - 1),), jnp.float32),
            pltpu.VMEM(((NUM_BINS + 1) * SHAPE,), jnp.float32),
            pltpu.VMEM((NUM_BINS * SHAPE,), jnp.float32),
            pltpu.VMEM(((NUM_BINS + 1) * SHAPE,), jnp.float32),
            pltpu.VMEM(((NUM_BINS + 1) * SHAPE,), jnp.float32),
            pltpu.VMEM((NUM_BINS * SHAPE,), jnp.float32),
            pltpu.VMEM((NUM_BINS * SHAPE,), jnp.float32),
            pltpu.VMEM((BLK, SHAPE), jnp.float32),
            pltpu.VMEM((BLK, SHAPE), jnp.float32),
            pltpu.VMEM((BLK * L,), jnp.float32),
            pltpu.VMEM((ROWS_W,), jnp.float32),
        ],
    )
    return f(inputs,
             unnormalized_widths.reshape(-1),
             unnormalized_heights.reshape(-1),
             unnormalized_derivatives.reshape(-1))
